# baseline (speedup 1.0000x reference)
import functools

import jax
import jax.numpy as jnp
from jax import lax
from jax.experimental import pallas as pl
from jax.experimental.pallas import tpu as pltpu
from jax.experimental.pallas import tpu_sc as plsc

B = 1024
F32 = jnp.float32


def _shift_rows(v, s):
    if s == 0:
        return v
    z = jnp.zeros((abs(s), v.shape[1]), v.dtype)
    if s > 0:
        return jnp.concatenate([v[s:], z], axis=0)
    return jnp.concatenate([z, v[:s]], axis=0)


def _im2col(v, H, W):
    A, C = v.shape
    r = lax.broadcasted_iota(jnp.int32, (A, 1), 0)
    hw = r % (H * W)
    h = hw // W
    w = hw % W
    cols = []
    for dy in (-1, 0, 1):
        for dx in (-1, 0, 1):
            sh = _shift_rows(v, dy * W + dx)
            m = (h + dy >= 0) & (h + dy < H) & (w + dx >= 0) & (w + dx < W)
            cols.append(jnp.where(m, sh, jnp.zeros_like(sh)))
    return jnp.concatenate(cols, axis=1)


def _pool2x2(v, N, H, W):
    A, C = v.shape
    t = jnp.max(v.reshape(A // 2, 2, C), axis=1)
    W2 = W // 2
    t = t.reshape(N * H // 2, 2 * W2, C)
    t = jnp.maximum(t[:, :W2, :], t[:, W2:, :])
    return t.reshape(N * (H // 2) * W2, C)


def _up2x2(v, N, H, W):
    A, C = v.shape
    t = v.reshape(N * H, W, C)
    t = jnp.concatenate([t, t], axis=1).reshape(N * H * 2 * W, C)
    t = jnp.broadcast_to(t[:, None, :], (N * H * 2 * W, 2, C))
    return t.reshape(N * H * 2 * W * 2, C)


def _k1_body(xc_ref, w1_ref, b1_ref, o_ref):
    u = jnp.dot(xc_ref[...], w1_ref[...], preferred_element_type=F32) + b1_ref[...]
    u = jnp.maximum(u, 0.0)
    o_ref[...] = _pool2x2(u, u.shape[0] // 576, 24, 24)


def _k2_body(a1_ref, w2_ref, b2_ref, w3_ref, b3_ref, perm_ref, efw_ref,
             efb_ref, cbt_ref, idx_ref):
    v = a1_ref[...]
    Nb = v.shape[0] // 144
    a2 = jnp.dot(_im2col(v, 12, 12), w2_ref[...],
                 preferred_element_type=F32) + b2_ref[...]
    a2 = jnp.maximum(a2, 0.0)
    p2 = _pool2x2(a2, Nb, 12, 12)
    a3 = jnp.dot(_im2col(p2, 6, 6), w3_ref[...],
                 preferred_element_type=F32) + b3_ref[...]
    a3 = jnp.maximum(a3, 0.0)
    p3 = _pool2x2(a3, Nb, 6, 6)
    v3 = p3.reshape(Nb, 9, 128)
    hcat = jnp.concatenate([v3[:, p, :] for p in range(9)], axis=1)
    hcp = jnp.dot(hcat, perm_ref[...], preferred_element_type=F32,
                  precision=lax.Precision.HIGHEST)
    z = jnp.dot(hcp, efw_ref[...], preferred_element_type=F32) + efb_ref[...]
    cbt = cbt_ref[...]
    cn = jnp.sum(cbt * cbt, axis=0, keepdims=True)
    s = jnp.dot(z, cbt, preferred_element_type=F32,
                precision=lax.Precision.HIGHEST) * (-2.0) + cn
    mn = jnp.min(s, axis=1, keepdims=True)
    ii = lax.broadcasted_iota(jnp.int32, s.shape, 1)
    idx = jnp.min(jnp.where(s <= mn, ii, jnp.int32(2 ** 30)),
                  axis=1, keepdims=True)
    idx_ref[...] = idx


def _k3_body(zq_ref, dfw_ref, dfb_ref, wc1_ref, bc1_ref, wc2_ref, bc2_ref,
             wc3_ref, bc3_ref, o_ref):
    zq = zq_ref[...]
    Nb = zq.shape[0]
    g = jnp.dot(zq, dfw_ref[...], preferred_element_type=F32) + dfb_ref[...]
    g3 = jnp.concatenate([g[:, p * 128:(p + 1) * 128][:, None, :]
                          for p in range(9)], axis=1)
    g = g3.reshape(Nb * 9, 128)
    d1 = jnp.dot(_im2col(g, 3, 3), wc1_ref[...],
                 preferred_element_type=F32) + bc1_ref[...]
    d1 = jnp.maximum(d1, 0.0)
    d1 = _up2x2(d1, Nb, 3, 3)
    d2 = jnp.dot(_im2col(d1, 6, 6), wc2_ref[...],
                 preferred_element_type=F32) + bc2_ref[...]
    d2 = jnp.maximum(d2, 0.0)
    d2 = _up2x2(d2, Nb, 6, 6)
    d3 = jnp.dot(_im2col(d2, 12, 12), wc3_ref[...],
                 preferred_element_type=F32) + bc3_ref[...]
    o_ref[...] = jnp.maximum(d3, 0.0)


def _sc_gather(codebook, idx):
    info = plsc.get_sparse_core_info()
    nw = info.num_cores * info.num_subcores
    bpw = B // nw
    mesh = plsc.VectorSubcoreMesh(core_axis_name="c", subcore_axis_name="s")

    @functools.partial(
        pl.kernel, mesh=mesh,
        out_type=jax.ShapeDtypeStruct((B, 128), F32),
        scratch_types=[
            pltpu.VMEM((bpw,), jnp.int32),
            pltpu.VMEM((bpw, 128), F32),
            pltpu.SemaphoreType.DMA,
        ],
    )
    def k(table_hbm, idx_hbm, out_hbm, idx_v, rows_v, sem):
        wid = lax.axis_index("s") * info.num_cores + lax.axis_index("c")
        base = wid * bpw
        pltpu.sync_copy(idx_hbm.at[pl.ds(base, bpw)], idx_v)
        pltpu.async_copy(table_hbm.at[idx_v], rows_v, sem).wait()
        pltpu.sync_copy(rows_v, out_hbm.at[pl.ds(base, bpw)])

    return k(codebook, idx)


def _conv_cat(w):
    return jnp.transpose(w, (2, 3, 1, 0)).reshape(9 * w.shape[1], w.shape[0])


def _convt_cat(w):
    wc = jnp.transpose(jnp.flip(w, (2, 3)), (1, 0, 2, 3))
    return _conv_cat(wc)


def kernel(x, ew1, eb1, ew2, eb2, ew3, eb3, efw, efb,
           dfw, dfb, dw1, db1, dw2, db2, dw3, db3, codebook):
    f = lambda a: a.astype(F32)

    w1c, b1 = _conv_cat(f(ew1)), f(eb1)[None, :]
    w2c, b2 = _conv_cat(f(ew2)), f(eb2)[None, :]
    w3c, b3 = _conv_cat(f(ew3)), f(eb3)[None, :]
    pid = jnp.arange(1152)
    perm = jnp.zeros((1152, 1152), F32).at[pid, (pid % 128) * 9 + pid // 128].set(1.0)
    efwt = jnp.transpose(f(efw), (1, 0))
    efb2 = f(efb)[None, :]
    dfw2 = jnp.transpose(f(dfw).reshape(128, 9, 128), (1, 0, 2)).reshape(1152, 128).T
    dfb2 = jnp.transpose(f(dfb).reshape(128, 9), (1, 0)).reshape(1, 1152)
    wc1, bc1 = _convt_cat(f(dw1)), f(db1)[None, :]
    wc2, bc2 = _convt_cat(f(dw2)), f(db2)[None, :]
    wc3 = jnp.tile(_convt_cat(f(dw3)), (1, 4))
    bc3 = jnp.tile(f(db3)[None, :], (1, 4))
    cbt = jnp.transpose(f(codebook), (1, 0))

    xp = jnp.pad(x.reshape(B, 24, 24), ((0, 0), (1, 1), (1, 1)))
    xcol = jnp.stack([xp[:, ky:ky + 24, kx:kx + 24]
                      for ky in range(3) for kx in range(3)],
                     axis=-1).reshape(B * 576, 9)

    full = lambda shp: pl.BlockSpec(shp, lambda i: tuple(0 for _ in shp))

    nb1 = 32
    a1p = pl.pallas_call(
        _k1_body,
        grid=(B // nb1,),
        in_specs=[pl.BlockSpec((nb1 * 576, 9), lambda i: (i, 0)),
                  full((9, 32)), full((1, 32))],
        out_specs=pl.BlockSpec((nb1 * 144, 32), lambda i: (i, 0)),
        out_shape=jax.ShapeDtypeStruct((B * 144, 32), F32),
    )(xcol, w1c, b1)

    nb2 = 32
    idx = pl.pallas_call(
        _k2_body,
        grid=(B // nb2,),
        in_specs=[pl.BlockSpec((nb2 * 144, 32), lambda i: (i, 0)),
                  full((288, 64)), full((1, 64)),
                  full((576, 128)), full((1, 128)),
                  full((1152, 1152)),
                  full((1152, 128)), full((1, 128)),
                  full((128, 1024))],
        out_specs=pl.BlockSpec((nb2, 1), lambda i: (i, 0)),
        out_shape=jax.ShapeDtypeStruct((B, 1), jnp.int32),
    )(a1p, w2c, b2, w3c, b3, perm, efwt, efb2, cbt)

    zq = _sc_gather(f(codebook), idx.reshape(B))

    nb3 = 32
    out4 = pl.pallas_call(
        _k3_body,
        grid=(B // nb3,),
        in_specs=[pl.BlockSpec((nb3, 128), lambda i: (i, 0)),
                  full((128, 1152)), full((1, 1152)),
                  full((1152, 64)), full((1, 64)),
                  full((576, 32)), full((1, 32)),
                  full((288, 4)), full((1, 4))],
        out_specs=pl.BlockSpec((nb3 * 144, 4), lambda i: (i, 0)),
        out_shape=jax.ShapeDtypeStruct((B * 144, 4), F32),
    )(zq, dfw2, dfb2, wc1, bc1, wc2, bc2, wc3, bc3)

    out = out4.reshape(B, 12, 12, 2, 2)
    out = jnp.transpose(out, (0, 1, 3, 2, 4)).reshape(B, 1, 24, 24)
    return out


# speedup vs baseline: 1.1543x; 1.1543x over previous
import functools

import jax
import jax.numpy as jnp
from jax import lax
from jax.experimental import pallas as pl
from jax.experimental.pallas import tpu as pltpu
from jax.experimental.pallas import tpu_sc as plsc

B = 1024
F32 = jnp.float32


def _shift_rows(v, s):
    if s == 0:
        return v
    z = jnp.zeros((abs(s), v.shape[1]), v.dtype)
    if s > 0:
        return jnp.concatenate([v[s:], z], axis=0)
    return jnp.concatenate([z, v[:s]], axis=0)


def _im2col(v, H, W):
    A, C = v.shape
    r = lax.broadcasted_iota(jnp.int32, (A, 1), 0)
    hw = r % (H * W)
    h = hw // W
    w = hw % W
    zero = jnp.zeros_like(v)
    mh = {-1: h >= 1, 1: h <= H - 2}
    mw = {-1: w >= 1, 1: w <= W - 2}
    cols = []
    for dy in (-1, 0, 1):
        u = _shift_rows(v, dy * W)
        if dy:
            u = jnp.where(mh[dy], u, zero)
        for dx in (-1, 0, 1):
            sh = _shift_rows(u, dx)
            cols.append(jnp.where(mw[dx], sh, zero) if dx else sh)
    return jnp.concatenate(cols, axis=1)


def _pool2x2(v, N, H, W):
    A, C = v.shape
    t = jnp.max(v.reshape(A // 2, 2, C), axis=1)
    W2 = W // 2
    t = t.reshape(N * H // 2, 2 * W2, C)
    t = jnp.maximum(t[:, :W2, :], t[:, W2:, :])
    return t.reshape(N * (H // 2) * W2, C)


def _up2x2(v, N, H, W):
    A, C = v.shape
    t = v.reshape(N * H, W, C)
    t = jnp.concatenate([t, t], axis=1).reshape(N * H * 2 * W, C)
    t = jnp.broadcast_to(t[:, None, :], (N * H * 2 * W, 2, C))
    return t.reshape(N * H * 2 * W * 2, C)


def _k1_body(xc_ref, w1_ref, b1_ref, o_ref):
    u = jnp.dot(xc_ref[...], w1_ref[...], preferred_element_type=F32) + b1_ref[...]
    u = jnp.maximum(u, 0.0)
    o_ref[...] = _pool2x2(u, u.shape[0] // 576, 24, 24)


def _k2_body(a1_ref, w2_ref, b2_ref, w3_ref, b3_ref, perm_ref, efw_ref,
             efb_ref, cbt_ref, idx_ref):
    v = a1_ref[...]
    Nb = v.shape[0] // 144
    a2 = jnp.dot(_im2col(v, 12, 12), w2_ref[...],
                 preferred_element_type=F32) + b2_ref[...]
    a2 = jnp.maximum(a2, 0.0)
    p2 = _pool2x2(a2, Nb, 12, 12)
    a3 = jnp.dot(_im2col(p2, 6, 6), w3_ref[...],
                 preferred_element_type=F32) + b3_ref[...]
    a3 = jnp.maximum(a3, 0.0)
    p3 = _pool2x2(a3, Nb, 6, 6)
    v3 = p3.reshape(Nb, 9, 128)
    hcat = jnp.concatenate([v3[:, p, :] for p in range(9)], axis=1)
    hcp = jnp.dot(hcat, perm_ref[...], preferred_element_type=F32,
                  precision=lax.Precision.HIGHEST)
    z = jnp.dot(hcp, efw_ref[...], preferred_element_type=F32) + efb_ref[...]
    cbt = cbt_ref[...]
    cn = jnp.sum(cbt * cbt, axis=0, keepdims=True)
    s = jnp.dot(z, cbt, preferred_element_type=F32,
                precision=lax.Precision.HIGHEST) * (-2.0) + cn
    mn = jnp.min(s, axis=1, keepdims=True)
    ii = lax.broadcasted_iota(jnp.int32, s.shape, 1)
    idx = jnp.min(jnp.where(s <= mn, ii, jnp.int32(2 ** 30)),
                  axis=1, keepdims=True)
    idx_ref[...] = idx


def _k3_body(zq_ref, dfw_ref, dfb_ref, wc1_ref, bc1_ref, wc2_ref, bc2_ref,
             wc3_ref, bc3_ref, o_ref):
    zq = zq_ref[...]
    Nb = zq.shape[0]
    g = jnp.dot(zq, dfw_ref[...], preferred_element_type=F32) + dfb_ref[...]
    g3 = jnp.concatenate([g[:, p * 128:(p + 1) * 128][:, None, :]
                          for p in range(9)], axis=1)
    g = g3.reshape(Nb * 9, 128)
    d1 = jnp.dot(_im2col(g, 3, 3), wc1_ref[...],
                 preferred_element_type=F32) + bc1_ref[...]
    d1 = jnp.maximum(d1, 0.0)
    d1 = _up2x2(d1, Nb, 3, 3)
    d2 = jnp.dot(_im2col(d1, 6, 6), wc2_ref[...],
                 preferred_element_type=F32) + bc2_ref[...]
    d2 = jnp.maximum(d2, 0.0)
    d2 = _up2x2(d2, Nb, 6, 6)
    d3 = jnp.dot(_im2col(d2, 12, 12), wc3_ref[...],
                 preferred_element_type=F32) + bc3_ref[...]
    o_ref[...] = jnp.maximum(d3, 0.0)


def _sc_gather(codebook, idx):
    info = plsc.get_sparse_core_info()
    nw = info.num_cores * info.num_subcores
    bpw = B // nw
    mesh = plsc.VectorSubcoreMesh(core_axis_name="c", subcore_axis_name="s")

    @functools.partial(
        pl.kernel, mesh=mesh,
        out_type=jax.ShapeDtypeStruct((B, 128), F32),
        scratch_types=[
            pltpu.VMEM((bpw,), jnp.int32),
            pltpu.VMEM((bpw, 128), F32),
            pltpu.SemaphoreType.DMA,
        ],
    )
    def k(table_hbm, idx_hbm, out_hbm, idx_v, rows_v, sem):
        wid = lax.axis_index("s") * info.num_cores + lax.axis_index("c")
        base = wid * bpw
        pltpu.sync_copy(idx_hbm.at[pl.ds(base, bpw)], idx_v)
        pltpu.async_copy(table_hbm.at[idx_v], rows_v, sem).wait()
        pltpu.sync_copy(rows_v, out_hbm.at[pl.ds(base, bpw)])

    return k(codebook, idx)


def _conv_cat(w):
    return jnp.transpose(w, (2, 3, 1, 0)).reshape(9 * w.shape[1], w.shape[0])


def _convt_cat(w):
    wc = jnp.transpose(jnp.flip(w, (2, 3)), (1, 0, 2, 3))
    return _conv_cat(wc)


def kernel(x, ew1, eb1, ew2, eb2, ew3, eb3, efw, efb,
           dfw, dfb, dw1, db1, dw2, db2, dw3, db3, codebook):
    f = lambda a: a.astype(F32)

    w1c, b1 = _conv_cat(f(ew1)), f(eb1)[None, :]
    w2c, b2 = _conv_cat(f(ew2)), f(eb2)[None, :]
    w3c, b3 = _conv_cat(f(ew3)), f(eb3)[None, :]
    pid = jnp.arange(1152)
    perm = jnp.zeros((1152, 1152), F32).at[pid, (pid % 128) * 9 + pid // 128].set(1.0)
    efwt = jnp.transpose(f(efw), (1, 0))
    efb2 = f(efb)[None, :]
    dfw2 = jnp.transpose(f(dfw).reshape(128, 9, 128), (1, 0, 2)).reshape(1152, 128).T
    dfb2 = jnp.transpose(f(dfb).reshape(128, 9), (1, 0)).reshape(1, 1152)
    wc1, bc1 = _convt_cat(f(dw1)), f(db1)[None, :]
    wc2, bc2 = _convt_cat(f(dw2)), f(db2)[None, :]
    wc3 = jnp.tile(_convt_cat(f(dw3)), (1, 4))
    bc3 = jnp.tile(f(db3)[None, :], (1, 4))
    cbt = jnp.transpose(f(codebook), (1, 0))

    xp = jnp.pad(x.reshape(B, 24, 24), ((0, 0), (1, 1), (1, 1)))
    xcol = jnp.stack([xp[:, ky:ky + 24, kx:kx + 24]
                      for ky in range(3) for kx in range(3)],
                     axis=-1).reshape(B * 576, 9)

    full = lambda shp: pl.BlockSpec(shp, lambda i: tuple(0 for _ in shp))

    nb1 = 32
    a1p = pl.pallas_call(
        _k1_body,
        grid=(B // nb1,),
        in_specs=[pl.BlockSpec((nb1 * 576, 9), lambda i: (i, 0)),
                  full((9, 32)), full((1, 32))],
        out_specs=pl.BlockSpec((nb1 * 144, 32), lambda i: (i, 0)),
        out_shape=jax.ShapeDtypeStruct((B * 144, 32), F32),
    )(xcol, w1c, b1)

    nb2 = 64
    idx = pl.pallas_call(
        _k2_body,
        grid=(B // nb2,),
        in_specs=[pl.BlockSpec((nb2 * 144, 32), lambda i: (i, 0)),
                  full((288, 64)), full((1, 64)),
                  full((576, 128)), full((1, 128)),
                  full((1152, 1152)),
                  full((1152, 128)), full((1, 128)),
                  full((128, 1024))],
        out_specs=pl.BlockSpec((nb2, 1), lambda i: (i, 0)),
        out_shape=jax.ShapeDtypeStruct((B, 1), jnp.int32),
    )(a1p, w2c, b2, w3c, b3, perm, efwt, efb2, cbt)

    zq = _sc_gather(f(codebook), idx.reshape(B))

    nb3 = 64
    out4 = pl.pallas_call(
        _k3_body,
        grid=(B // nb3,),
        in_specs=[pl.BlockSpec((nb3, 128), lambda i: (i, 0)),
                  full((128, 1152)), full((1, 1152)),
                  full((1152, 64)), full((1, 64)),
                  full((576, 32)), full((1, 32)),
                  full((288, 4)), full((1, 4))],
        out_specs=pl.BlockSpec((nb3 * 144, 4), lambda i: (i, 0)),
        out_shape=jax.ShapeDtypeStruct((B * 144, 4), F32),
    )(zq, dfw2, dfb2, wc1, bc1, wc2, bc2, wc3, bc3)

    out = out4.reshape(B, 12, 12, 2, 2)
    out = jnp.transpose(out, (0, 1, 3, 2, 4)).reshape(B, 1, 24, 24)
    return out
